# baseline (speedup 1.0000x reference)
import functools

import jax
import jax.numpy as jnp
from jax import lax
from jax.experimental import pallas as pl
from jax.experimental.pallas import tpu as pltpu
from jax.experimental.pallas import tpu_sc as plsc

_B, _S = 4096, 50
_V, _D = 10, 512
_N = _B * _S
_C = 80
_NCHUNK = _N // _C
_NW = 32
_CPW = _NCHUNK // _NW
_NBUF = 2
_R = 64


def _sc_gather(idx3d, table_rep):
    mesh = plsc.VectorSubcoreMesh(core_axis_name="c", subcore_axis_name="s")

    @functools.partial(
        pl.kernel,
        mesh=mesh,
        out_type=jax.ShapeDtypeStruct((_N, _D), jnp.float32),
        scratch_types=[
            pltpu.VMEM((_CPW, _C), jnp.int32),
        ]
        + [pltpu.VMEM((_C, _D), jnp.float32) for _ in range(_NBUF)]
        + [pltpu.SemaphoreType.DMA for _ in range(2 * _NBUF)],
    )
    def k(idx_hbm, table_hbm, out_hbm, idx_v, *bufs_and_sems):
        bufs = bufs_and_sems[:_NBUF]
        gsem = bufs_and_sems[_NBUF:2 * _NBUF]
        ssem = bufs_and_sems[2 * _NBUF:]
        wid = lax.axis_index("s") * 2 + lax.axis_index("c")
        pltpu.sync_copy(idx_hbm.at[wid], idx_v)

        def fire_gather(j, b):
            pltpu.async_copy(table_hbm.at[idx_v.at[j]], bufs[b], gsem[b])

        def wait_gather(j, b):
            pltpu.make_async_copy(
                table_hbm.at[idx_v.at[j]], bufs[b], gsem[b]).wait()

        def fire_scatter(j, b):
            row0 = (wid * _CPW + j) * _C
            pltpu.async_copy(bufs[b], out_hbm.at[pl.ds(row0, _C)], ssem[b])

        def wait_scatter(j, b):
            row0 = (wid * _CPW + j) * _C
            pltpu.make_async_copy(
                bufs[b], out_hbm.at[pl.ds(row0, _C)], ssem[b]).wait()

        for b in range(_NBUF):
            fire_gather(b, b)

        def outer(o, carry):
            for b in range(_NBUF):
                j = o * _NBUF + b
                wait_gather(j, b)
                fire_scatter(j, b)
            for b in range(_NBUF):
                j = o * _NBUF + b
                wait_scatter(j, b)
                fire_gather(j + _NBUF, b)
            return carry

        lax.fori_loop(0, _CPW // _NBUF - 1, outer, 0)

        for b in range(_NBUF):
            j = _CPW - _NBUF + b
            wait_gather(j, b)
            fire_scatter(j, b)
        for b in range(_NBUF):
            j = _CPW - _NBUF + b
            wait_scatter(j, b)

    return k(idx3d, table_rep)


def kernel(x, table):
    idx = x.astype(jnp.int32).reshape(-1)
    idx = idx + _V * (jnp.arange(_N, dtype=jnp.int32) % _R)
    idx3d = idx.reshape(_NW, _CPW, _C)
    table_rep = jnp.tile(table, (_R, 1))
    out = _sc_gather(idx3d, table_rep)
    return out.reshape(_B, _S, _D)


# speedup vs baseline: 1.0375x; 1.0375x over previous
import functools

import jax
import jax.numpy as jnp
from jax import lax
from jax.experimental import pallas as pl
from jax.experimental.pallas import tpu as pltpu
from jax.experimental.pallas import tpu_sc as plsc

_B, _S = 4096, 50
_V, _D = 10, 512
_N = _B * _S
_C = 40
_NCHUNK = _N // _C
_NW = 32
_CPW = _NCHUNK // _NW
_NBUF = 4
_R = 128


def _sc_gather(idx3d, table_rep):
    mesh = plsc.VectorSubcoreMesh(core_axis_name="c", subcore_axis_name="s")

    @functools.partial(
        pl.kernel,
        mesh=mesh,
        out_type=jax.ShapeDtypeStruct((_N, _D), jnp.float32),
        scratch_types=[
            pltpu.VMEM((_CPW, _C), jnp.int32),
        ]
        + [pltpu.VMEM((_C, _D), jnp.float32) for _ in range(_NBUF)]
        + [pltpu.SemaphoreType.DMA for _ in range(2 * _NBUF)],
    )
    def k(idx_hbm, table_hbm, out_hbm, idx_v, *bufs_and_sems):
        bufs = bufs_and_sems[:_NBUF]
        gsem = bufs_and_sems[_NBUF:2 * _NBUF]
        ssem = bufs_and_sems[2 * _NBUF:]
        wid = lax.axis_index("s") * 2 + lax.axis_index("c")
        pltpu.sync_copy(idx_hbm.at[wid], idx_v)

        def fire_gather(j, b):
            pltpu.async_copy(table_hbm.at[idx_v.at[j]], bufs[b], gsem[b])

        def wait_gather(j, b):
            pltpu.make_async_copy(
                table_hbm.at[idx_v.at[j]], bufs[b], gsem[b]).wait()

        def fire_scatter(j, b):
            row0 = (wid * _CPW + j) * _C
            pltpu.async_copy(bufs[b], out_hbm.at[pl.ds(row0, _C)], ssem[b])

        def wait_scatter(j, b):
            row0 = (wid * _CPW + j) * _C
            pltpu.make_async_copy(
                bufs[b], out_hbm.at[pl.ds(row0, _C)], ssem[b]).wait()

        for b in range(_NBUF):
            fire_gather(b, b)

        def outer(o, carry):
            for b in range(_NBUF):
                j = o * _NBUF + b
                wait_gather(j, b)
                fire_scatter(j, b)
            for b in range(_NBUF):
                j = o * _NBUF + b
                wait_scatter(j, b)
                fire_gather(j + _NBUF, b)
            return carry

        lax.fori_loop(0, _CPW // _NBUF - 1, outer, 0)

        for b in range(_NBUF):
            j = _CPW - _NBUF + b
            wait_gather(j, b)
            fire_scatter(j, b)
        for b in range(_NBUF):
            j = _CPW - _NBUF + b
            wait_scatter(j, b)

    return k(idx3d, table_rep)


def kernel(x, table):
    idx = x.astype(jnp.int32).reshape(-1)
    idx = idx + _V * (jnp.arange(_N, dtype=jnp.int32) % _R)
    idx3d = idx.reshape(_NW, _CPW, _C)
    table_rep = jnp.tile(table, (_R, 1))
    out = _sc_gather(idx3d, table_rep)
    return out.reshape(_B, _S, _D)
